# baseline (speedup 1.0000x reference)
import functools

import jax
import jax.numpy as jnp
from jax import lax
from jax.experimental import pallas as pl
from jax.experimental.pallas import tpu as pltpu
from jax.experimental.pallas import tpu_sc as plsc

N = 100000
P = 150
F = 4
L = 16
NC, NS = 2, 16
NW = NC * NS
G = 4
CHUNK_ROWS = G * L
CHUNKS = 49
ROWS_PER_TILE = CHUNK_ROWS * CHUNKS
LAST_BASE = N - ROWS_PER_TILE
ROW_W = P * F
CHUNK_W = CHUNK_ROWS * ROW_W


def _rsqrt(x, iters):
    i = lax.bitcast_convert_type(x, jnp.int32)
    i = jnp.int32(0x5F3759DF) - (i >> 1)
    y = lax.bitcast_convert_type(i, jnp.float32)
    for _ in range(iters):
        y = y * (1.5 - 0.5 * x * y * y)
    return y


def _sc_body(lf_hbm, pt_hbm, out_hbm, sbuf, obuf, abuf, bbuf, sem_a, sem_b):
    wid = lax.axis_index("s") * NC + lax.axis_index("c")
    base = jnp.where(wid < NW - 1, wid * ROWS_PER_TILE, LAST_BASE)

    pltpu.sync_copy(pt_hbm.at[pl.ds(base * 2, ROWS_PER_TILE * 2)], sbuf)

    iota = lax.iota(jnp.int32, L)
    lane600 = iota * ROW_W
    lane2 = iota * 2

    def issue(c, buf, sem):
        start = (base + c * CHUNK_ROWS) * ROW_W
        pltpu.async_copy(lf_hbm.at[pl.ds(start, CHUNK_W)], buf, sem)

    def wait(buf, sem):
        pltpu.make_async_copy(lf_hbm.at[pl.ds(0, CHUNK_W)], buf, sem).wait()

    def process(c, buf):
        ixb = [lane600 + (v * L * ROW_W) for v in range(G)]
        S = [plsc.load_gather(sbuf, [lane2 + ((c * CHUNK_ROWS + v * L) * 2)])
             for v in range(G)]
        px = [plsc.load_gather(buf, [ixb[v]]) for v in range(G)]
        py = [plsc.load_gather(buf, [ixb[v] + 1]) for v in range(G)]
        zf = jnp.zeros((L,), jnp.float32)
        zi = jnp.zeros((L,), jnp.int32)

        def step(j, carry):
            ix, px, py, dist, db, idxv, jm1 = carry
            ix = [ix[v] + F for v in range(G)]
            x = [plsc.load_gather(buf, [ix[v]]) for v in range(G)]
            y = [plsc.load_gather(buf, [ix[v] + 1]) for v in range(G)]
            ndist, ndb, nidx = [], [], []
            for v in range(G):
                dx = x[v] - px[v]
                dy = y[v] - py[v]
                ssq = jnp.maximum(dx * dx + dy * dy, 1e-30)
                ln = ssq * _rsqrt(ssq, 2)
                dn = dist[v] + ln
                fire = (S[v] >= dist[v]) & (S[v] < dn)
                ndb.append(jnp.where(fire, dist[v], db[v]))
                nidx.append(jnp.where(fire, jm1, idxv[v]))
                ndist.append(dn)
            return (ix, x, y, ndist, ndb, nidx, jm1 + 1)

        carry = ([ixb[v] for v in range(G)], px, py,
                 [zf] * G, [zf] * G, [zi] * G, zi)
        _, _, _, dist, db, idxv, _ = lax.fori_loop(1, P, step, carry)

        for v in range(G):
            ge = S[v] >= dist[v]
            iv = jnp.where(ge, P - 2, idxv[v])
            gi = ixb[v] + iv * F
            pbx = plsc.load_gather(buf, [gi])
            pby = plsc.load_gather(buf, [gi + 1])
            pax = plsc.load_gather(buf, [gi + F])
            pay = plsc.load_gather(buf, [gi + F + 1])
            vx = pax - pbx
            vy = pay - pby
            m2 = jnp.maximum(vx * vx + vy * vy, 1e-30)
            r = _rsqrt(m2, 3)
            dbv = jnp.where(ge, dist[v] - m2 * r, db[v])
            t = (S[v] - dbv) * r
            lr2 = lane2 + ((c * CHUNK_ROWS + v * L) * 2)
            plsc.store_scatter(obuf, [lr2], pbx + t * vx)
            plsc.store_scatter(obuf, [lr2 + 1], pby + t * vy)

    issue(0, abuf, sem_a)

    def outer(i, _):
        c = i * 2

        @pl.when(c + 1 < CHUNKS)
        def _():
            issue(c + 1, bbuf, sem_b)

        wait(abuf, sem_a)
        process(c, abuf)

        @pl.when(c + 2 < CHUNKS)
        def _():
            issue(c + 2, abuf, sem_a)

        @pl.when(c + 1 < CHUNKS)
        def _():
            wait(bbuf, sem_b)
            process(c + 1, bbuf)

        return 0

    lax.fori_loop(0, (CHUNKS + 1) // 2, outer, 0)

    pltpu.sync_copy(obuf, out_hbm.at[pl.ds(base * 2, ROWS_PER_TILE * 2)])


@jax.jit
def _sl_to_xy(lf_flat, pt_flat):
    mesh = plsc.VectorSubcoreMesh(core_axis_name="c", subcore_axis_name="s",
                                  num_cores=NC, num_subcores=NS)
    run = functools.partial(
        pl.kernel,
        out_type=jax.ShapeDtypeStruct((N * 2,), jnp.float32),
        mesh=mesh,
        compiler_params=pltpu.CompilerParams(needs_layout_passes=False),
        scratch_types=[
            pltpu.VMEM((ROWS_PER_TILE * 2,), jnp.float32),
            pltpu.VMEM((ROWS_PER_TILE * 2,), jnp.float32),
            pltpu.VMEM((CHUNK_W,), jnp.float32),
            pltpu.VMEM((CHUNK_W,), jnp.float32),
            pltpu.SemaphoreType.DMA,
            pltpu.SemaphoreType.DMA,
        ],
    )(_sc_body)
    return run(lf_flat, pt_flat)


def kernel(lane_features, pt_sl):
    out = _sl_to_xy(lane_features.reshape(-1), pt_sl.reshape(-1))
    return out.reshape(N, 2)


# speedup vs baseline: 103.9362x; 103.9362x over previous
import functools

import jax
import jax.numpy as jnp
from jax import lax
from jax.experimental import pallas as pl
from jax.experimental.pallas import tpu as pltpu
from jax.experimental.pallas import tpu_sc as plsc

N = 100000
P = 150
F = 4
L = 16
NC, NS = 2, 16
NW = NC * NS
RT = 3200
NG = RT // L
LAST_BASE = 96896
LAST_STORE = N - LAST_BASE
UNROLL = 2


def _rsqrt(x, iters):
    i = lax.bitcast_convert_type(x, jnp.int32)
    i = jnp.int32(0x5F3759DF) - (i >> 1)
    y = lax.bitcast_convert_type(i, jnp.float32)
    for _ in range(iters):
        y = y * (1.5 - 0.5 * x * y * y)
    return y


def _sc_body(lft, ptT, xout, yout, sbuf, dist, db, pbx, pby, pax, pay,
             obx, oby, b0, b1, b2, s0, s1, s2):
    wid = lax.axis_index("s") * NC + lax.axis_index("c")
    last = wid == NW - 1
    base = jnp.where(last, LAST_BASE, wid * RT)

    pltpu.sync_copy(ptT.at[:, pl.ds(base, RT)], sbuf)

    bufs = [b0, b1, b2]
    sems = [s0, s1, s2]
    iota = lax.iota(jnp.int32, L)
    zf = jnp.zeros((L,), jnp.float32)

    def issue(j, slot):
        pltpu.async_copy(lft.at[j, :, pl.ds(base, RT)], bufs[slot],
                         sems[slot])

    def wait(slot):
        pltpu.make_async_copy(
            lft.at[0, :, pl.ds(0, RT)], bufs[slot], sems[slot]).wait()

    def step(cur, prv, first=False, last_pt=False, unroll=UNROLL):
        @plsc.parallel_loop(0, NG, unroll=unroll)
        def _(g):
            sl = pl.ds(g * L, L)
            x = bufs[cur][0, sl]
            y = bufs[cur][1, sl]
            px = bufs[prv][0, sl]
            py = bufs[prv][1, sl]
            S = sbuf[0, sl]
            d0 = zf if first else dist[sl]
            dx = x - px
            dy = y - py
            ssq = jnp.maximum(dx * dx + dy * dy, 1e-30)
            ln = ssq * _rsqrt(ssq, 2)
            dn = d0 + ln
            fire = (S >= d0) & (S < dn)
            u = fire | (S >= dn) if last_pt else fire
            if not last_pt:
                dist[sl] = dn
            idxs = iota + g * L
            plsc.store_scatter(db, [idxs], d0, mask=u)
            plsc.store_scatter(pbx, [idxs], px, mask=u)
            plsc.store_scatter(pby, [idxs], py, mask=u)
            plsc.store_scatter(pax, [idxs], x, mask=u)
            plsc.store_scatter(pay, [idxs], y, mask=u)

    issue(0, 0)
    issue(1, 1)
    issue(2, 2)
    wait(0)
    wait(1)
    step(1, 0, first=True)
    issue(3, 0)

    def outer(i, _):
        for k in range(3):
            j = 3 * i + 2 + k
            slot = (2 + k) % 3
            prv = (1 + k) % 3
            wait(slot)
            step(slot, prv)
            issue(j + 2, (k + 1) % 3)
        return 0

    lax.fori_loop(0, 48, outer, 0)

    for j in (146, 147, 148, 149):
        slot = j % 3
        wait(slot)
        step(slot, (j - 1) % 3, last_pt=(j == 149), unroll=1)
        if j + 2 <= 149:
            issue(j + 2, (j + 2) % 3)

    @plsc.parallel_loop(0, NG, unroll=1)
    def _(g):
        sl = pl.ds(g * L, L)
        S = sbuf[0, sl]
        d = db[sl]
        bx = pbx[sl]
        by = pby[sl]
        vx = pax[sl] - bx
        vy = pay[sl] - by
        m2 = jnp.maximum(vx * vx + vy * vy, 1e-30)
        r = _rsqrt(m2, 3)
        t = (S - d) * r
        obx[sl] = bx + t * vx
        oby[sl] = by + t * vy

    @pl.when(jnp.logical_not(last))
    def _():
        pltpu.sync_copy(obx, xout.at[pl.ds(base, RT)])
        pltpu.sync_copy(oby, yout.at[pl.ds(base, RT)])

    @pl.when(last)
    def _():
        pltpu.sync_copy(obx.at[pl.ds(0, LAST_STORE)],
                        xout.at[pl.ds(LAST_BASE, LAST_STORE)])
        pltpu.sync_copy(oby.at[pl.ds(0, LAST_STORE)],
                        yout.at[pl.ds(LAST_BASE, LAST_STORE)])


@jax.jit
def _sl_to_xy(lft, ptT):
    mesh = plsc.VectorSubcoreMesh(core_axis_name="c", subcore_axis_name="s",
                                  num_cores=NC, num_subcores=NS)
    run = functools.partial(
        pl.kernel,
        out_type=(jax.ShapeDtypeStruct((N,), jnp.float32),
                  jax.ShapeDtypeStruct((N,), jnp.float32)),
        mesh=mesh,
        compiler_params=pltpu.CompilerParams(needs_layout_passes=False),
        scratch_types=(
            [pltpu.VMEM((2, RT), jnp.float32)]
            + [pltpu.VMEM((RT,), jnp.float32) for _ in range(8)]
            + [pltpu.VMEM((F, RT), jnp.float32) for _ in range(3)]
            + [pltpu.SemaphoreType.DMA] * 3
        ),
    )(_sc_body)
    return run(lft, ptT)


def kernel(lane_features, pt_sl):
    lft = lane_features.transpose(1, 2, 0)
    ptT = pt_sl.T
    X, Y = _sl_to_xy(lft, ptT)
    return jnp.stack([X, Y], axis=1)
